# baseline (speedup 1.0000x reference)
import dataclasses
import functools

import jax
import jax.numpy as jnp
from jax import lax
from jax.experimental import pallas as pl
from jax.experimental.pallas import tpu as pltpu
from jax.experimental.pallas import tpu_sc as plsc

_P = 5.0
_A = 10.0

_NC = 2
_NS = 16
_W = 3200


def _sc_segment_sums(x, pos, neg, n_pad):
    v_nodes = x.shape[0]
    e_edges = pos.shape[1]
    assert e_edges % _W == 0
    total_chunks = e_edges // _W
    bins_per_tile = n_pad // _NS

    mesh = plsc.VectorSubcoreMesh(core_axis_name="c", subcore_axis_name="s")
    cp = pltpu.CompilerParams()
    if "needs_layout_passes" in pltpu.CompilerParams.__dataclass_fields__:
        cp = dataclasses.replace(cp, needs_layout_passes=False)

    @functools.partial(
        pl.kernel,
        out_type=(
            jax.ShapeDtypeStruct((_NC * n_pad,), jnp.float32),
            jax.ShapeDtypeStruct((_NC * n_pad,), jnp.float32),
        ),
        mesh=mesh,
        compiler_params=cp,
        scratch_types=[
            pltpu.VMEM((v_nodes,), jnp.float32),
        ] + [pltpu.VMEM((_W,), jnp.int32) for _ in range(6)]
          + [pltpu.VMEM((_W,), jnp.float32) for _ in range(6)]
          + [
            pltpu.VMEM((bins_per_tile,), jnp.float32),
            pltpu.VMEM_SHARED((n_pad,), jnp.float32),
            pltpu.VMEM_SHARED((n_pad,), jnp.float32),
            pltpu.SemaphoreType.DMA,
            pltpu.SemaphoreType.DMA,
        ],
    )
    def k(x_hbm, pos_hbm, neg_hbm, num_out, den_out, x_v,
          ci0, ci1, ci2, vi0, vi1, vi2, nb0, nb1, nb2, eb0, eb1, eb2,
          zbuf, num_sh, den_sh, sem, sem_in):
        bufs = ((ci0, vi0, nb0, eb0), (ci1, vi1, nb1, eb1),
                (ci2, vi2, nb2, eb2))
        c = lax.axis_index("c")
        s = lax.axis_index("s")

        @plsc.parallel_loop(0, bins_per_tile, step=16, unroll=4)
        def _(i):
            zbuf[pl.ds(i, 16)] = jnp.zeros((16,), jnp.float32)

        pltpu.sync_copy(zbuf, num_sh.at[pl.ds(s * bins_per_tile, bins_per_tile)])
        pltpu.sync_copy(zbuf, den_sh.at[pl.ds(s * bins_per_tile, bins_per_tile)])

        pltpu.sync_copy(x_hbm, x_v)
        plsc.subcore_barrier()

        nch = lax.div(total_chunks - 1 - s, _NS) + 1

        def process(adj_hbm, is_neg):
            def base(kk):
                return (s + kk * _NS) * _W

            def fire_in(kk, m):
                b = base(kk)
                pltpu.async_copy(adj_hbm.at[0, pl.ds(b, _W)],
                                 bufs[m][0], sem_in)
                pltpu.async_copy(adj_hbm.at[1, pl.ds(b, _W)],
                                 bufs[m][1], sem_in)

            def wait_in(kk, m):
                b = base(kk)
                pltpu.make_async_copy(adj_hbm.at[0, pl.ds(b, _W)],
                                      bufs[m][0], sem_in).wait()
                pltpu.make_async_copy(adj_hbm.at[1, pl.ds(b, _W)],
                                      bufs[m][1], sem_in).wait()

            def drain_scatters(m):
                ci, _, nb, eb = bufs[m]
                pltpu.make_async_copy(nb, num_sh.at[ci], sem).wait()
                pltpu.make_async_copy(eb, den_sh.at[ci], sem).wait()

            def compute_and_fire(m, kk):
                ci, vi_b, nb, eb = bufs[m]

                @plsc.parallel_loop(0, _W, step=16, unroll=8)
                def _(i):
                    vi = vi_b[pl.ds(i, 16)]
                    xg = plsc.load_gather(x_v, [vi])
                    lit = (1.0 - xg) if is_neg else xg
                    e = jnp.exp(lit * _P)
                    nb[pl.ds(i, 16)] = lit * e
                    eb[pl.ds(i, 16)] = e

                pltpu.async_copy(nb, num_sh.at[ci], sem, add=True)
                pltpu.async_copy(eb, den_sh.at[ci], sem, add=True)

            fire_in(0, 0)

            @pl.loop(0, nch + 2, step=3)
            def _(kk):
                for p in range(3):
                    m, m1 = p, (p + 1) % 3
                    j = kk + p

                    @pl.when((j >= 2) & (j - 2 < nch))
                    def _():
                        drain_scatters(m1)

                    @pl.when(j + 1 < nch)
                    def _():
                        fire_in(j + 1, m1)

                    @pl.when(j < nch)
                    def _():
                        wait_in(j, m)
                        compute_and_fire(m, j)

        @pl.when(c == 0)
        def _():
            process(pos_hbm, False)

        @pl.when(c == 1)
        def _():
            process(neg_hbm, True)

        plsc.subcore_barrier()
        base = c * n_pad + s * bins_per_tile
        pltpu.sync_copy(num_sh.at[pl.ds(s * bins_per_tile, bins_per_tile)], zbuf)
        pltpu.sync_copy(zbuf, num_out.at[pl.ds(base, bins_per_tile)])
        pltpu.sync_copy(den_sh.at[pl.ds(s * bins_per_tile, bins_per_tile)], zbuf)
        pltpu.sync_copy(zbuf, den_out.at[pl.ds(base, bins_per_tile)])

    return k(x, pos, neg)


def _tc_loss(num_flat, den_flat, n_pad, num_clauses):
    rows = n_pad // 128

    def body(n_ref, d_ref, o_ref):
        n = (n_ref[pl.ds(0, n_pad)] + n_ref[pl.ds(n_pad, n_pad)]).reshape(
            rows, 128)
        d = (d_ref[pl.ds(0, n_pad)] + d_ref[pl.ds(n_pad, n_pad)]).reshape(
            rows, 128)
        r = n / d
        sm = 1.0 / (1.0 + jnp.exp(_A * (0.5 - r)))
        idx = (lax.broadcasted_iota(jnp.int32, (rows, 128), 0) * 128
               + lax.broadcasted_iota(jnp.int32, (rows, 128), 1))
        term = jnp.where(idx < num_clauses, jnp.log(sm), 0.0)
        o_ref[0, 0] = -jnp.sum(term)

    out = pl.pallas_call(
        body,
        out_shape=jax.ShapeDtypeStruct((1, 1), jnp.float32),
        out_specs=pl.BlockSpec(memory_space=pltpu.SMEM),
    )(num_flat, den_flat)
    return out[0, 0]


def kernel(xv, adj_pos, adj_neg):
    x = xv.reshape(-1)
    v_nodes = x.shape[0]
    num_clauses = v_nodes
    e_edges = adj_pos.shape[1]
    assert adj_neg.shape[1] == e_edges
    assert e_edges % _W == 0

    n_pad = ((num_clauses + _NS * 16 - 1) // (_NS * 16)) * (_NS * 16)
    if n_pad == num_clauses:
        n_pad += _NS * 16

    num_flat, den_flat = _sc_segment_sums(x, adj_pos, adj_neg, n_pad)
    return _tc_loss(num_flat, den_flat, n_pad, num_clauses)


# speedup vs baseline: 1.0092x; 1.0092x over previous
import dataclasses
import functools

import jax
import jax.numpy as jnp
from jax import lax
from jax.experimental import pallas as pl
from jax.experimental.pallas import tpu as pltpu
from jax.experimental.pallas import tpu_sc as plsc

_P = 5.0
_A = 10.0

_NC = 2
_NS = 16
_W = 1280


def _sc_segment_sums(x, pos, neg, n_pad):
    v_nodes = x.shape[0]
    e_edges = pos.shape[1]
    assert e_edges % _W == 0
    total_chunks = e_edges // _W
    bins_per_tile = n_pad // _NS

    mesh = plsc.VectorSubcoreMesh(core_axis_name="c", subcore_axis_name="s")
    cp = pltpu.CompilerParams()
    if "needs_layout_passes" in pltpu.CompilerParams.__dataclass_fields__:
        cp = dataclasses.replace(cp, needs_layout_passes=False)

    @functools.partial(
        pl.kernel,
        out_type=(
            jax.ShapeDtypeStruct((_NC * n_pad,), jnp.float32),
            jax.ShapeDtypeStruct((_NC * n_pad,), jnp.float32),
        ),
        mesh=mesh,
        compiler_params=cp,
        scratch_types=[
            pltpu.VMEM((v_nodes,), jnp.float32),
        ] + [pltpu.VMEM((_W,), jnp.int32) for _ in range(6)]
          + [pltpu.VMEM((_W,), jnp.float32) for _ in range(6)]
          + [
            pltpu.VMEM((bins_per_tile,), jnp.float32),
            pltpu.VMEM_SHARED((n_pad,), jnp.float32),
            pltpu.VMEM_SHARED((n_pad,), jnp.float32),
            pltpu.SemaphoreType.DMA,
            pltpu.SemaphoreType.DMA,
        ],
    )
    def k(x_hbm, pos_hbm, neg_hbm, num_out, den_out, x_v,
          ci0, ci1, ci2, vi0, vi1, vi2, nb0, nb1, nb2, eb0, eb1, eb2,
          zbuf, num_sh, den_sh, sem, sem_in):
        bufs = ((ci0, vi0, nb0, eb0), (ci1, vi1, nb1, eb1),
                (ci2, vi2, nb2, eb2))
        c = lax.axis_index("c")
        s = lax.axis_index("s")

        @plsc.parallel_loop(0, bins_per_tile, step=16, unroll=4)
        def _(i):
            zbuf[pl.ds(i, 16)] = jnp.zeros((16,), jnp.float32)

        pltpu.sync_copy(zbuf, num_sh.at[pl.ds(s * bins_per_tile, bins_per_tile)])
        pltpu.sync_copy(zbuf, den_sh.at[pl.ds(s * bins_per_tile, bins_per_tile)])

        pltpu.sync_copy(x_hbm, x_v)
        plsc.subcore_barrier()

        nch = lax.div(total_chunks - 1 - s, _NS) + 1

        def process(adj_hbm, is_neg):
            def base(kk):
                return (s + kk * _NS) * _W

            def fire_in(kk, m):
                b = base(kk)
                pltpu.async_copy(adj_hbm.at[0, pl.ds(b, _W)],
                                 bufs[m][0], sem_in)
                pltpu.async_copy(adj_hbm.at[1, pl.ds(b, _W)],
                                 bufs[m][1], sem_in)

            def wait_in(kk, m):
                b = base(kk)
                pltpu.make_async_copy(adj_hbm.at[0, pl.ds(b, _W)],
                                      bufs[m][0], sem_in).wait()
                pltpu.make_async_copy(adj_hbm.at[1, pl.ds(b, _W)],
                                      bufs[m][1], sem_in).wait()

            def drain_scatters(m):
                ci, _, nb, eb = bufs[m]
                pltpu.make_async_copy(nb, num_sh.at[ci], sem).wait()
                pltpu.make_async_copy(eb, den_sh.at[ci], sem).wait()

            def compute_and_fire(m, kk):
                ci, vi_b, nb, eb = bufs[m]

                @plsc.parallel_loop(0, _W, step=16, unroll=8)
                def _(i):
                    vi = vi_b[pl.ds(i, 16)]
                    xg = plsc.load_gather(x_v, [vi])
                    lit = (1.0 - xg) if is_neg else xg
                    e = jnp.exp(lit * _P)
                    nb[pl.ds(i, 16)] = lit * e
                    eb[pl.ds(i, 16)] = e

                pltpu.async_copy(nb, num_sh.at[ci], sem, add=True)
                pltpu.async_copy(eb, den_sh.at[ci], sem, add=True)

            fire_in(0, 0)

            @pl.loop(0, nch + 2, step=3)
            def _(kk):
                for p in range(3):
                    m, m1 = p, (p + 1) % 3
                    j = kk + p

                    @pl.when((j >= 2) & (j - 2 < nch))
                    def _():
                        drain_scatters(m1)

                    @pl.when(j + 1 < nch)
                    def _():
                        fire_in(j + 1, m1)

                    @pl.when(j < nch)
                    def _():
                        wait_in(j, m)
                        compute_and_fire(m, j)

        @pl.when(c == 0)
        def _():
            process(pos_hbm, False)

        @pl.when(c == 1)
        def _():
            process(neg_hbm, True)

        plsc.subcore_barrier()
        base = c * n_pad + s * bins_per_tile
        pltpu.sync_copy(num_sh.at[pl.ds(s * bins_per_tile, bins_per_tile)], zbuf)
        pltpu.sync_copy(zbuf, num_out.at[pl.ds(base, bins_per_tile)])
        pltpu.sync_copy(den_sh.at[pl.ds(s * bins_per_tile, bins_per_tile)], zbuf)
        pltpu.sync_copy(zbuf, den_out.at[pl.ds(base, bins_per_tile)])

    return k(x, pos, neg)


def _tc_loss(num_flat, den_flat, n_pad, num_clauses):
    rows = n_pad // 128

    def body(n_ref, d_ref, o_ref):
        n = (n_ref[pl.ds(0, n_pad)] + n_ref[pl.ds(n_pad, n_pad)]).reshape(
            rows, 128)
        d = (d_ref[pl.ds(0, n_pad)] + d_ref[pl.ds(n_pad, n_pad)]).reshape(
            rows, 128)
        r = n / d
        sm = 1.0 / (1.0 + jnp.exp(_A * (0.5 - r)))
        idx = (lax.broadcasted_iota(jnp.int32, (rows, 128), 0) * 128
               + lax.broadcasted_iota(jnp.int32, (rows, 128), 1))
        term = jnp.where(idx < num_clauses, jnp.log(sm), 0.0)
        o_ref[0, 0] = -jnp.sum(term)

    out = pl.pallas_call(
        body,
        out_shape=jax.ShapeDtypeStruct((1, 1), jnp.float32),
        out_specs=pl.BlockSpec(memory_space=pltpu.SMEM),
    )(num_flat, den_flat)
    return out[0, 0]


def kernel(xv, adj_pos, adj_neg):
    x = xv.reshape(-1)
    v_nodes = x.shape[0]
    num_clauses = v_nodes
    e_edges = adj_pos.shape[1]
    assert adj_neg.shape[1] == e_edges
    assert e_edges % _W == 0

    n_pad = ((num_clauses + _NS * 16 - 1) // (_NS * 16)) * (_NS * 16)
    if n_pad == num_clauses:
        n_pad += _NS * 16

    num_flat, den_flat = _sc_segment_sums(x, adj_pos, adj_neg, n_pad)
    return _tc_loss(num_flat, den_flat, n_pad, num_clauses)
